# baseline (speedup 1.0000x reference)
import functools
import jax
import jax.numpy as jnp
from jax import lax
from jax.experimental import pallas as pl
from jax.experimental.pallas import tpu as pltpu
from jax.experimental.pallas import tpu_sc as plsc

N = 50000
D = 512
G = 256
B = 2000
NB = N // B

NW = 32
SEG_PER_W = G // NW
RB = 80
NBUF = N // RB
SB = RB // 16
FB = RB * D
ACC = SEG_PER_W * D
NC16 = D // 16

_NEG = -1e30


def _gate_kernel(feat, seg, wg, bg, gate, cnt_out):
    i = pl.program_id(0)

    @pl.when(i == 0)
    def _():
        cnt_out[...] = jnp.zeros_like(cnt_out)

    x = feat[...]
    g = jnp.dot(x, wg[...], preferred_element_type=jnp.float32) + bg[0, 0]
    gate[...] = g
    s = seg[0, 0, :]
    ids = lax.broadcasted_iota(jnp.int32, (B, G), 1)
    oh = s[:, None] == ids
    cnt_out[0, :] = cnt_out[0, :] + jnp.sum(oh.astype(jnp.float32), axis=0)


def _sc_pool_kernel(feat_hbm, g_hbm, seg_hbm, starts_hbm, out_hbm, den_hbm,
                    sv, fbuf, gbuf, sbuf, acc, dvm, sem0, sem1):
    c = lax.axis_index("c")
    s = lax.axis_index("s")
    w = s * 2 + c
    lo8 = w * SEG_PER_W
    pltpu.sync_copy(starts_hbm.at[pl.ds(lo8, 16)], sv)
    svv = sv[...]
    lo = svv[0]
    hi = svv[SEG_PER_W]
    b0 = lo // RB
    b1 = (hi + RB - 1) // RB

    zeros16 = jnp.zeros((16,), jnp.float32)
    for k in range(ACC // 16):
        acc[pl.ds(16 * k, 16)] = zeros16

    lane = lax.iota(jnp.int32, 16)

    def _issue(i, par, sem):
        pltpu.async_copy(feat_hbm.at[pl.ds(i * RB, RB), :],
                         fbuf.at[pl.ds(par * RB, RB), :], sem)
        pltpu.async_copy(g_hbm.at[pl.ds(i * RB, RB)],
                         gbuf.at[pl.ds(par * RB, RB)], sem)
        pltpu.async_copy(seg_hbm.at[pl.ds(i * RB, RB)],
                         sbuf.at[pl.ds(par * RB, RB)], sem)

    def _drain(i, par, sem):
        pltpu.make_async_copy(feat_hbm.at[pl.ds(i * RB, RB), :],
                              fbuf.at[pl.ds(par * RB, RB), :], sem).wait()
        pltpu.make_async_copy(g_hbm.at[pl.ds(i * RB, RB)],
                              gbuf.at[pl.ds(par * RB, RB)], sem).wait()
        pltpu.make_async_copy(seg_hbm.at[pl.ds(i * RB, RB)],
                              sbuf.at[pl.ds(par * RB, RB)], sem).wait()

    @pl.when(b0 < b1)
    def _():
        _issue(b0, 0, sem0)

    def buf_body(i, carry):
        par = lax.rem(i - b0, 2)
        nxt = i + 1

        @pl.when(nxt < b1)
        def _():
            @pl.when(par == 0)
            def _():
                _issue(nxt, 1, sem1)

            @pl.when(par == 1)
            def _():
                _issue(nxt, 0, sem0)

        @pl.when(par == 0)
        def _():
            _drain(i, 0, sem0)

        @pl.when(par == 1)
        def _():
            _drain(i, 1, sem1)

        def sb_body(sb, carry2):
            run2, m, sd, dvec = carry2[0], carry2[1], carry2[2], carry2[3]
            a = list(carry2[4:])
            rbase = par * RB + sb * 16
            svec = sbuf[pl.ds(rbase, 16)]
            gvec = gbuf[pl.ds(rbase, 16)]
            for r in range(16):
                s_r = svec[r]
                g_r = gvec[r]
                ok = (s_r >= lo8) & (s_r < lo8 + SEG_PER_W)
                eff = jnp.where(ok, s_r, -1)
                changed = eff != run2
                do_flush = changed & (run2 >= 0)
                abase = (run2 - lo8) * D
                ivec = 1.0 / jnp.full((16,), sd)

                @pl.when(do_flush)
                def _():
                    for j in range(NC16):
                        acc[pl.ds(abase + 16 * j, 16)] = a[j] * ivec

                dvec = jnp.where(do_flush & (lane == run2 - lo8),
                                 jnp.full((16,), sd), dvec)

                m_new = jnp.where(changed, g_r, jnp.maximum(m, g_r))
                f_arg = jnp.where(changed, 0.0, m - m_new)
                fvec = jnp.exp(jnp.full((16,), f_arg))
                pvec = jnp.exp(jnp.full((16,), g_r - m_new))
                f0 = fvec[0]
                p0 = pvec[0]
                sd = jnp.where(changed, 1.0, sd * f0 + p0)
                factor = jnp.where(changed & ok, 0.0,
                                   jnp.where(ok, f0, 1.0))
                facv = jnp.full((16,), factor)
                pev = jnp.where(ok, pvec, jnp.zeros((16,), jnp.float32))
                row = rbase + r
                a = [a[j] * facv + pev * fbuf[row, pl.ds(16 * j, 16)]
                     for j in range(NC16)]
                run2 = eff
                m = m_new
            return tuple([run2, m, sd, dvec] + a)

        return lax.fori_loop(0, SB, sb_body, carry)

    init = tuple([jnp.int32(-1), jnp.float32(0.0), jnp.float32(1.0),
                  jnp.zeros((16,), jnp.float32)] + [zeros16] * NC16)
    fin = lax.fori_loop(b0, b1, buf_body, init)
    run_f, s_f, dvec_f = fin[0], fin[2], fin[3]
    abase_f = (run_f - lo8) * D
    iv_f = 1.0 / jnp.full((16,), s_f)

    @pl.when(run_f >= 0)
    def _():
        for j in range(NC16):
            acc[pl.ds(abase_f + 16 * j, 16)] = fin[4 + j] * iv_f

    dvec_f = jnp.where((run_f >= 0) & (lane == run_f - lo8),
                       jnp.full((16,), s_f), dvec_f)
    dvm[...] = dvec_f
    pltpu.sync_copy(acc, out_hbm.at[pl.ds(w * ACC, ACC)])
    pltpu.sync_copy(dvm, den_hbm.at[pl.ds(w * 16, 16)])


def _final_kernel(pooled, den, wn, bn, out):
    msk = den[0, :][:, None] > 0.0
    out[...] = jnp.dot(pooled[...], wn[...],
                       preferred_element_type=jnp.float32) + \
        jnp.where(msk, bn[...], 0.0)


def kernel(feat, segment_ids, Wg, bg, Wn, bn):
    seg32 = segment_ids.astype(jnp.int32)
    seg3 = seg32.reshape(NB, 1, B)
    bg2 = bg.reshape(1, 1)
    bn2 = bn.reshape(1, D)

    gate, cnt = pl.pallas_call(
        _gate_kernel,
        grid=(NB,),
        in_specs=[
            pl.BlockSpec((B, D), lambda i: (i, 0)),
            pl.BlockSpec((1, 1, B), lambda i: (i, 0, 0)),
            pl.BlockSpec((D, 1), lambda i: (0, 0)),
            pl.BlockSpec((1, 1), lambda i: (0, 0)),
        ],
        out_specs=[
            pl.BlockSpec((B, 1), lambda i: (i, 0)),
            pl.BlockSpec((1, G), lambda i: (0, 0)),
        ],
        out_shape=[
            jax.ShapeDtypeStruct((N, 1), jnp.float32),
            jax.ShapeDtypeStruct((1, G), jnp.float32),
        ],
    )(feat, seg3, Wg, bg2)

    cnti = cnt[0].astype(jnp.int32)
    starts = jnp.concatenate(
        [jnp.zeros((1,), jnp.int32), jnp.cumsum(cnti, dtype=jnp.int32)])
    starts = jnp.pad(starts, (0, 15), constant_values=N)

    sc_pool = pl.kernel(
        _sc_pool_kernel,
        out_type=[
            jax.ShapeDtypeStruct((G * D,), jnp.float32),
            jax.ShapeDtypeStruct((NW * 16,), jnp.float32),
        ],
        mesh=plsc.VectorSubcoreMesh(core_axis_name="c", subcore_axis_name="s"),
        scratch_types=[
            pltpu.VMEM((16,), jnp.int32),
            pltpu.VMEM((2 * RB, D), jnp.float32),
            pltpu.VMEM((2 * RB,), jnp.float32),
            pltpu.VMEM((2 * RB,), jnp.int32),
            pltpu.VMEM((ACC,), jnp.float32),
            pltpu.VMEM((16,), jnp.float32),
            pltpu.SemaphoreType.DMA,
            pltpu.SemaphoreType.DMA,
        ],
        compiler_params=pltpu.CompilerParams(needs_layout_passes=False),
    )
    pooled_flat, den_raw = sc_pool(feat, gate.reshape(N), seg32, starts)
    pooled = pooled_flat.reshape(G, D)
    den = den_raw.reshape(NW, 16)[:, :SEG_PER_W].reshape(1, G)

    out = pl.pallas_call(
        _final_kernel,
        in_specs=[
            pl.BlockSpec((G, D), lambda: (0, 0)),
            pl.BlockSpec((1, G), lambda: (0, 0)),
            pl.BlockSpec((D, D), lambda: (0, 0)),
            pl.BlockSpec((1, D), lambda: (0, 0)),
        ],
        out_specs=pl.BlockSpec((G, D), lambda: (0, 0)),
        out_shape=jax.ShapeDtypeStruct((G, D), jnp.float32),
    )(pooled, den, Wn, bn2)

    return out


# speedup vs baseline: 1.9437x; 1.9437x over previous
import functools
import jax
import jax.numpy as jnp
from jax import lax
from jax.experimental import pallas as pl
from jax.experimental.pallas import tpu as pltpu
from jax.experimental.pallas import tpu_sc as plsc

N = 50000
D = 512
G = 256
B = 2000
NB = N // B

NW = 32
SEG_PER_W = G // NW
RB = 80
NBUF = N // RB
SB = RB // 16
FB = RB * D
ACC = SEG_PER_W * D
NC16 = D // 16
BPB = B // RB

_NEG = -1e30


def _gate_kernel(feat, seg, wg, bg, p_out, cnt_out, corr_out, m_scr):
    i = pl.program_id(0)

    @pl.when(i == 0)
    def _():
        cnt_out[...] = jnp.zeros_like(cnt_out)
        m_scr[...] = jnp.full_like(m_scr, _NEG)

    x = feat[...]
    g = jnp.dot(x, wg[...], preferred_element_type=jnp.float32) + bg[0, 0]
    lm = jnp.max(g)
    p_out[...] = jnp.exp(g - lm)
    ids32 = lax.broadcasted_iota(jnp.int32, (1, 32), 1)
    m_scr[0, :] = jnp.where(ids32[0, :] == i, lm, m_scr[0, :])
    s = seg[0, 0, :]
    ids = lax.broadcasted_iota(jnp.int32, (B, G), 1)
    oh = s[:, None] == ids
    cnt_out[0, :] = cnt_out[0, :] + jnp.sum(oh.astype(jnp.float32), axis=0)

    @pl.when(i == NB - 1)
    def _():
        mv = m_scr[0, :]
        mx = jnp.max(mv)
        corr_out[0, :] = jnp.exp(jnp.minimum(mv - mx, 0.0))


def _sc_pool_kernel(feat_hbm, p_hbm, seg_hbm, starts_hbm, corr_hbm,
                    out_hbm, den_hbm,
                    sv, fbuf, pbuf, sbuf, acc, cb, dvm, sem0, sem1):
    c = lax.axis_index("c")
    s = lax.axis_index("s")
    w = s * 2 + c
    lo8 = w * SEG_PER_W
    pltpu.sync_copy(starts_hbm.at[pl.ds(lo8, 16)], sv)
    pltpu.sync_copy(corr_hbm, cb)
    svv = sv[...]
    lo = svv[0]
    hi = svv[SEG_PER_W]
    b0 = lo // RB
    b1 = (hi + RB - 1) // RB
    c0 = cb[pl.ds(0, 16)]
    c1 = cb[pl.ds(16, 16)]
    lane = lax.iota(jnp.int32, 16)

    zeros16 = jnp.zeros((16,), jnp.float32)
    for k in range(ACC // 16):
        acc[pl.ds(16 * k, 16)] = zeros16

    def _issue(i, par, sem):
        pltpu.async_copy(feat_hbm.at[pl.ds(i * RB, RB), :],
                         fbuf.at[pl.ds(par * RB, RB), :], sem)
        pltpu.async_copy(p_hbm.at[pl.ds(i * RB, RB)],
                         pbuf.at[pl.ds(par * RB, RB)], sem)
        pltpu.async_copy(seg_hbm.at[pl.ds(i * RB, RB)],
                         sbuf.at[pl.ds(par * RB, RB)], sem)

    def _drain(i, par, sem):
        pltpu.make_async_copy(feat_hbm.at[pl.ds(i * RB, RB), :],
                              fbuf.at[pl.ds(par * RB, RB), :], sem).wait()
        pltpu.make_async_copy(p_hbm.at[pl.ds(i * RB, RB)],
                              pbuf.at[pl.ds(par * RB, RB)], sem).wait()
        pltpu.make_async_copy(seg_hbm.at[pl.ds(i * RB, RB)],
                              sbuf.at[pl.ds(par * RB, RB)], sem).wait()

    @pl.when(b0 < b1)
    def _():
        _issue(b0, 0, sem0)

    def buf_body(i, carry):
        par = lax.rem(i - b0, 2)
        nxt = i + 1

        @pl.when(nxt < b1)
        def _():
            @pl.when(par == 0)
            def _():
                _issue(nxt, 1, sem1)

            @pl.when(par == 1)
            def _():
                _issue(nxt, 0, sem0)

        @pl.when(par == 0)
        def _():
            _drain(i, 0, sem0)

        @pl.when(par == 1)
        def _():
            _drain(i, 1, sem1)

        bi = jnp.full((16,), i // BPB)
        corr_spl = jnp.where(
            bi < 16,
            c0.at[jnp.clip(bi, 0, 15)].get(mode='promise_in_bounds'),
            c1.at[jnp.clip(bi - 16, 0, 15)].get(mode='promise_in_bounds'))

        def sb_body(sb, carry2):
            run2, sd, dvec = carry2[0], carry2[1], carry2[2]
            a = list(carry2[3:])
            rbase = par * RB + sb * 16
            svec = sbuf[pl.ds(rbase, 16)]
            pvec = pbuf[pl.ds(rbase, 16)] * corr_spl
            for r in range(16):
                s_r = svec[r]
                p_r = pvec[r]
                ok = (s_r >= lo8) & (s_r < lo8 + SEG_PER_W)
                eff = jnp.where(ok, s_r, -1)
                changed = eff != run2
                do_flush = changed & (run2 >= 0)
                abase = (run2 - lo8) * D

                @pl.when(do_flush)
                def _():
                    for j in range(NC16):
                        acc[pl.ds(abase + 16 * j, 16)] = a[j]

                dvec = jnp.where(do_flush & (lane == run2 - lo8),
                                 jnp.full((16,), sd), dvec)
                contrib = jnp.where(ok, p_r, 0.0)
                zf = changed & ok
                sd = jnp.where(zf, 0.0, sd) + contrib
                row = rbase + r
                a = [jnp.where(zf, 0.0, a[j]) +
                     contrib * fbuf[row, pl.ds(16 * j, 16)]
                     for j in range(NC16)]
                run2 = eff
            return tuple([run2, sd, dvec] + a)

        return lax.fori_loop(0, SB, sb_body, carry)

    init = tuple([jnp.int32(-1), jnp.float32(0.0), zeros16] +
                 [zeros16] * NC16)
    fin = lax.fori_loop(b0, b1, buf_body, init)
    run_f, sd_f, dvec_f = fin[0], fin[1], fin[2]
    abase_f = (run_f - lo8) * D

    @pl.when(run_f >= 0)
    def _():
        for j in range(NC16):
            acc[pl.ds(abase_f + 16 * j, 16)] = fin[3 + j]

    dvec_f = jnp.where((run_f >= 0) & (lane == run_f - lo8),
                       jnp.full((16,), sd_f), dvec_f)
    dvm[...] = dvec_f
    pltpu.sync_copy(acc, out_hbm.at[pl.ds(w * ACC, ACC)])
    pltpu.sync_copy(dvm, den_hbm.at[pl.ds(w * 16, 16)])


def _final_kernel(pooled, den, wn, bn, out):
    d = den[0, :][:, None]
    msk = d > 0.0
    inv = jnp.where(msk, 1.0 / jnp.where(msk, d, 1.0), 0.0)
    pn = pooled[...] * inv
    out[...] = jnp.dot(pn, wn[...], preferred_element_type=jnp.float32) + \
        jnp.where(msk, bn[...], 0.0)


def kernel(feat, segment_ids, Wg, bg, Wn, bn):
    seg32 = segment_ids.astype(jnp.int32)
    seg3 = seg32.reshape(NB, 1, B)
    bg2 = bg.reshape(1, 1)
    bn2 = bn.reshape(1, D)

    p, cnt, corr = pl.pallas_call(
        _gate_kernel,
        grid=(NB,),
        in_specs=[
            pl.BlockSpec((B, D), lambda i: (i, 0)),
            pl.BlockSpec((1, 1, B), lambda i: (i, 0, 0)),
            pl.BlockSpec((D, 1), lambda i: (0, 0)),
            pl.BlockSpec((1, 1), lambda i: (0, 0)),
        ],
        out_specs=[
            pl.BlockSpec((B, 1), lambda i: (i, 0)),
            pl.BlockSpec((1, G), lambda i: (0, 0)),
            pl.BlockSpec((1, 32), lambda i: (0, 0)),
        ],
        out_shape=[
            jax.ShapeDtypeStruct((N, 1), jnp.float32),
            jax.ShapeDtypeStruct((1, G), jnp.float32),
            jax.ShapeDtypeStruct((1, 32), jnp.float32),
        ],
        scratch_shapes=[pltpu.VMEM((1, 32), jnp.float32)],
    )(feat, seg3, Wg, bg2)

    cnti = cnt[0].astype(jnp.int32)
    starts = jnp.concatenate(
        [jnp.zeros((1,), jnp.int32), jnp.cumsum(cnti, dtype=jnp.int32)])
    starts = jnp.pad(starts, (0, 15), constant_values=N)

    sc_pool = pl.kernel(
        _sc_pool_kernel,
        out_type=[
            jax.ShapeDtypeStruct((G * D,), jnp.float32),
            jax.ShapeDtypeStruct((NW * 16,), jnp.float32),
        ],
        mesh=plsc.VectorSubcoreMesh(core_axis_name="c", subcore_axis_name="s"),
        scratch_types=[
            pltpu.VMEM((16,), jnp.int32),
            pltpu.VMEM((2 * RB, D), jnp.float32),
            pltpu.VMEM((2 * RB,), jnp.float32),
            pltpu.VMEM((2 * RB,), jnp.int32),
            pltpu.VMEM((ACC,), jnp.float32),
            pltpu.VMEM((32,), jnp.float32),
            pltpu.VMEM((16,), jnp.float32),
            pltpu.SemaphoreType.DMA,
            pltpu.SemaphoreType.DMA,
        ],
        compiler_params=pltpu.CompilerParams(needs_layout_passes=False),
    )
    pooled_flat, den_raw = sc_pool(feat, p.reshape(N), seg32, starts,
                                   corr.reshape(32))
    pooled = pooled_flat.reshape(G, D)
    den = den_raw.reshape(NW, 16)[:, :SEG_PER_W].reshape(1, G)

    out = pl.pallas_call(
        _final_kernel,
        in_specs=[
            pl.BlockSpec((G, D), lambda: (0, 0)),
            pl.BlockSpec((1, G), lambda: (0, 0)),
            pl.BlockSpec((D, D), lambda: (0, 0)),
            pl.BlockSpec((1, D), lambda: (0, 0)),
        ],
        out_specs=pl.BlockSpec((G, D), lambda: (0, 0)),
        out_shape=jax.ShapeDtypeStruct((G, D), jnp.float32),
    )(pooled, den, Wn, bn2)

    return out


# speedup vs baseline: 1.9441x; 1.0002x over previous
import functools
import jax
import jax.numpy as jnp
from jax import lax
from jax.experimental import pallas as pl
from jax.experimental.pallas import tpu as pltpu
from jax.experimental.pallas import tpu_sc as plsc

N = 50000
D = 512
G = 256
B = 2000
NB = N // B

NW = 32
SEG_PER_W = G // NW
RB = 80
NBUF = N // RB
SB = RB // 16
FB = RB * D
ACC = SEG_PER_W * D
NC16 = D // 16
BPB = B // RB

_NEG = -1e30


def _gate_kernel(feat, seg_s, wg, bg, p_out, cnt_out, corr_out, m_scr):
    i = pl.program_id(0)

    @pl.when(i == 0)
    def _():
        cnt_out[...] = jnp.zeros_like(cnt_out)
        m_scr[...] = jnp.full_like(m_scr, _NEG)

    x = feat[...]
    g = jnp.dot(x, wg[...], preferred_element_type=jnp.float32) + bg[0, 0]
    lm = jnp.max(g)
    p_out[...] = jnp.exp(g - lm)
    ids32 = lax.broadcasted_iota(jnp.int32, (1, 32), 1)
    m_scr[0, :] = jnp.where(ids32[0, :] == i, lm, m_scr[0, :])
    ss = seg_s[0, 0, :]
    ids = lax.broadcasted_iota(jnp.int32, (B // 16, G), 1)
    below = (ss[:, None] < ids).astype(jnp.float32)
    cnt_out[0, :] = cnt_out[0, :] + jnp.sum(below, axis=0)

    @pl.when(i == NB - 1)
    def _():
        mv = m_scr[0, :]
        mx = jnp.max(mv)
        corr_out[0, :] = jnp.exp(jnp.minimum(mv - mx, 0.0))


def _sc_pool_kernel(feat_hbm, p_hbm, seg_hbm, los_hbm, his_hbm, corr_hbm,
                    out_hbm, den_hbm,
                    sv, sv2, fbuf, pbuf, sbuf, acc, cb, dvm, sem0, sem1):
    c = lax.axis_index("c")
    s = lax.axis_index("s")
    w = s * 2 + c
    lo8 = w * SEG_PER_W
    pltpu.sync_copy(los_hbm.at[pl.ds(lo8, 16)], sv)
    pltpu.sync_copy(his_hbm.at[pl.ds(lo8, 16)], sv2)
    pltpu.sync_copy(corr_hbm, cb)
    lo = sv[...][0]
    hi = sv2[...][SEG_PER_W]
    b0 = lo // RB
    b1 = (hi + RB - 1) // RB
    c0 = cb[pl.ds(0, 16)]
    c1 = cb[pl.ds(16, 16)]
    lane = lax.iota(jnp.int32, 16)

    zeros16 = jnp.zeros((16,), jnp.float32)
    for k in range(ACC // 16):
        acc[pl.ds(16 * k, 16)] = zeros16

    def _issue(i, par, sem):
        pltpu.async_copy(feat_hbm.at[pl.ds(i * RB, RB), :],
                         fbuf.at[pl.ds(par * RB, RB), :], sem)
        pltpu.async_copy(p_hbm.at[pl.ds(i * RB, RB)],
                         pbuf.at[pl.ds(par * RB, RB)], sem)
        pltpu.async_copy(seg_hbm.at[pl.ds(i * RB, RB)],
                         sbuf.at[pl.ds(par * RB, RB)], sem)

    def _drain(i, par, sem):
        pltpu.make_async_copy(feat_hbm.at[pl.ds(i * RB, RB), :],
                              fbuf.at[pl.ds(par * RB, RB), :], sem).wait()
        pltpu.make_async_copy(p_hbm.at[pl.ds(i * RB, RB)],
                              pbuf.at[pl.ds(par * RB, RB)], sem).wait()
        pltpu.make_async_copy(seg_hbm.at[pl.ds(i * RB, RB)],
                              sbuf.at[pl.ds(par * RB, RB)], sem).wait()

    @pl.when(b0 < b1)
    def _():
        _issue(b0, 0, sem0)

    def buf_body(i, carry):
        par = lax.rem(i - b0, 2)
        nxt = i + 1

        @pl.when(nxt < b1)
        def _():
            @pl.when(par == 0)
            def _():
                _issue(nxt, 1, sem1)

            @pl.when(par == 1)
            def _():
                _issue(nxt, 0, sem0)

        @pl.when(par == 0)
        def _():
            _drain(i, 0, sem0)

        @pl.when(par == 1)
        def _():
            _drain(i, 1, sem1)

        bi = jnp.full((16,), i // BPB)
        corr_spl = jnp.where(
            bi < 16,
            c0.at[jnp.clip(bi, 0, 15)].get(mode='promise_in_bounds'),
            c1.at[jnp.clip(bi - 16, 0, 15)].get(mode='promise_in_bounds'))

        def sb_body(sb, carry2):
            run2, sd, dvec = carry2[0], carry2[1], carry2[2]
            a = list(carry2[3:])
            rbase = par * RB + sb * 16
            svec = sbuf[pl.ds(rbase, 16)]
            pvec = pbuf[pl.ds(rbase, 16)] * corr_spl
            for r in range(16):
                s_r = svec[r]
                p_r = pvec[r]
                ok = (s_r >= lo8) & (s_r < lo8 + SEG_PER_W)
                eff = jnp.where(ok, s_r, -1)
                changed = eff != run2
                do_flush = changed & (run2 >= 0)
                abase = (run2 - lo8) * D

                @pl.when(do_flush)
                def _():
                    for j in range(NC16):
                        acc[pl.ds(abase + 16 * j, 16)] = a[j]

                dvec = jnp.where(do_flush & (lane == run2 - lo8),
                                 jnp.full((16,), sd), dvec)
                contrib = jnp.where(ok, p_r, 0.0)
                zf = changed & ok
                sd = jnp.where(zf, 0.0, sd) + contrib
                row = rbase + r
                a = [jnp.where(zf, 0.0, a[j]) +
                     contrib * fbuf[row, pl.ds(16 * j, 16)]
                     for j in range(NC16)]
                run2 = eff
            return tuple([run2, sd, dvec] + a)

        return lax.fori_loop(0, SB, sb_body, carry)

    init = tuple([jnp.int32(-1), jnp.float32(0.0), zeros16] +
                 [zeros16] * NC16)
    fin = lax.fori_loop(b0, b1, buf_body, init)
    run_f, sd_f, dvec_f = fin[0], fin[1], fin[2]
    abase_f = (run_f - lo8) * D

    @pl.when(run_f >= 0)
    def _():
        for j in range(NC16):
            acc[pl.ds(abase_f + 16 * j, 16)] = fin[3 + j]

    dvec_f = jnp.where((run_f >= 0) & (lane == run_f - lo8),
                       jnp.full((16,), sd_f), dvec_f)
    dvm[...] = dvec_f
    pltpu.sync_copy(acc, out_hbm.at[pl.ds(w * ACC, ACC)])
    pltpu.sync_copy(dvm, den_hbm.at[pl.ds(w * 16, 16)])


def _final_kernel(pooled, den, wn, bn, out):
    d = den[0, :][:, None]
    msk = d > 0.0
    inv = jnp.where(msk, 1.0 / jnp.where(msk, d, 1.0), 0.0)
    pn = pooled[...] * inv
    out[...] = jnp.dot(pn, wn[...], preferred_element_type=jnp.float32) + \
        jnp.where(msk, bn[...], 0.0)


def kernel(feat, segment_ids, Wg, bg, Wn, bn):
    seg32 = segment_ids.astype(jnp.int32)
    seg_s = seg32[::16].reshape(NB, 1, B // 16)
    bg2 = bg.reshape(1, 1)
    bn2 = bn.reshape(1, D)

    p, cnt, corr = pl.pallas_call(
        _gate_kernel,
        grid=(NB,),
        in_specs=[
            pl.BlockSpec((B, D), lambda i: (i, 0)),
            pl.BlockSpec((1, 1, B // 16), lambda i: (i, 0, 0)),
            pl.BlockSpec((D, 1), lambda i: (0, 0)),
            pl.BlockSpec((1, 1), lambda i: (0, 0)),
        ],
        out_specs=[
            pl.BlockSpec((B, 1), lambda i: (i, 0)),
            pl.BlockSpec((1, G), lambda i: (0, 0)),
            pl.BlockSpec((1, 32), lambda i: (0, 0)),
        ],
        out_shape=[
            jax.ShapeDtypeStruct((N, 1), jnp.float32),
            jax.ShapeDtypeStruct((1, G), jnp.float32),
            jax.ShapeDtypeStruct((1, 32), jnp.float32),
        ],
        scratch_shapes=[pltpu.VMEM((1, 32), jnp.float32)],
    )(feat, seg_s, Wg, bg2)

    si = cnt[0].astype(jnp.int32)
    los = jnp.pad(jnp.maximum(16 * si - 16, 0), (0, 16), constant_values=N)
    his = jnp.pad(jnp.minimum(16 * si, N), (0, 16), constant_values=N)

    sc_pool = pl.kernel(
        _sc_pool_kernel,
        out_type=[
            jax.ShapeDtypeStruct((G * D,), jnp.float32),
            jax.ShapeDtypeStruct((NW * 16,), jnp.float32),
        ],
        mesh=plsc.VectorSubcoreMesh(core_axis_name="c", subcore_axis_name="s"),
        scratch_types=[
            pltpu.VMEM((16,), jnp.int32),
            pltpu.VMEM((16,), jnp.int32),
            pltpu.VMEM((2 * RB, D), jnp.float32),
            pltpu.VMEM((2 * RB,), jnp.float32),
            pltpu.VMEM((2 * RB,), jnp.int32),
            pltpu.VMEM((ACC,), jnp.float32),
            pltpu.VMEM((32,), jnp.float32),
            pltpu.VMEM((16,), jnp.float32),
            pltpu.SemaphoreType.DMA,
            pltpu.SemaphoreType.DMA,
        ],
        compiler_params=pltpu.CompilerParams(needs_layout_passes=False),
    )
    pooled_flat, den_raw = sc_pool(feat, p.reshape(N), seg32, los, his,
                                   corr.reshape(32))
    pooled = pooled_flat.reshape(G, D)
    den = den_raw.reshape(NW, 16)[:, :SEG_PER_W].reshape(1, G)

    out = pl.pallas_call(
        _final_kernel,
        in_specs=[
            pl.BlockSpec((G, D), lambda: (0, 0)),
            pl.BlockSpec((1, G), lambda: (0, 0)),
            pl.BlockSpec((D, D), lambda: (0, 0)),
            pl.BlockSpec((1, D), lambda: (0, 0)),
        ],
        out_specs=pl.BlockSpec((G, D), lambda: (0, 0)),
        out_shape=jax.ShapeDtypeStruct((G, D), jnp.float32),
    )(pooled, den, Wn, bn2)

    return out


# speedup vs baseline: 2.1092x; 1.0849x over previous
import functools
import jax
import jax.numpy as jnp
from jax import lax
from jax.experimental import pallas as pl
from jax.experimental.pallas import tpu as pltpu
from jax.experimental.pallas import tpu_sc as plsc

N = 50000
D = 512
G = 256
B = 2000
NB = N // B

NW = 32
GH = G // 2
SEG_PER_W = GH // NW
RB = 80
NBUF = N // RB
SB = RB // 16
FB = RB * D
ACC = SEG_PER_W * D
NC16 = D // 16
BPB = B // RB

_NEG = -1e30


def _gate_kernel(feat, seg_s, seg, wg, bg, p_out, cnt_out, corr_out,
                 pooled_lo, den_lo, m_scr, mrun):
    i = pl.program_id(0)

    @pl.when(i == 0)
    def _():
        cnt_out[...] = jnp.zeros_like(cnt_out)
        m_scr[...] = jnp.full_like(m_scr, _NEG)
        mrun[...] = jnp.full_like(mrun, _NEG)
        pooled_lo[...] = jnp.zeros_like(pooled_lo)
        den_lo[...] = jnp.zeros_like(den_lo)

    x = feat[...]
    g = jnp.dot(x, wg[...], preferred_element_type=jnp.float32) + bg[0, 0]
    lm = jnp.max(g)
    pv = jnp.exp(g - lm)
    p_out[...] = pv
    ids32 = lax.broadcasted_iota(jnp.int32, (1, 32), 1)
    m_scr[0, :] = jnp.where(ids32[0, :] == i, lm, m_scr[0, :])
    ss = seg_s[0, 0, :]
    ids = lax.broadcasted_iota(jnp.int32, (B // 16, G), 1)
    below = (ss[:, None] < ids).astype(jnp.float32)
    cnt_out[0, :] = cnt_out[0, :] + jnp.sum(below, axis=0)

    mo = mrun[0, :]
    mn = jnp.maximum(mo, lm)
    f_s = jnp.exp(mo - mn)[0]
    e_b = jnp.exp(lm - mn[0])
    s_full = seg[0, 0, :]
    ids_lo = lax.broadcasted_iota(jnp.int32, (B, GH), 1)
    oh_lo = (s_full[:, None] == ids_lo).astype(jnp.float32)
    pw = pv * e_b
    pooled_lo[...] = pooled_lo[...] * f_s + lax.dot_general(
        oh_lo, x * pw, dimension_numbers=(((0,), (0,)), ((), ())),
        preferred_element_type=jnp.float32)
    den_lo[0, :] = den_lo[0, :] * f_s + jnp.sum(oh_lo * pw, axis=0)
    mrun[0, :] = mn

    @pl.when(i == NB - 1)
    def _():
        mv = m_scr[0, :]
        mx = jnp.max(mv)
        corr_out[0, :] = jnp.exp(jnp.minimum(mv - mx, 0.0))


def _sc_pool_kernel(feat_hbm, p_hbm, seg_hbm, bounds_hbm, corr_hbm,
                    out_hbm, den_hbm,
                    sv, fbuf, pbuf, sbuf, acc, cb, dvm, sem0, sem1):
    c = lax.axis_index("c")
    s = lax.axis_index("s")
    w = s * 2 + c
    lo8 = GH + w * SEG_PER_W
    pltpu.sync_copy(bounds_hbm.at[pl.ds(w * 8, 16)], sv)
    pltpu.sync_copy(corr_hbm, cb)
    svv = sv[...]
    lo = svv[0]
    hi = svv[1]
    b0 = lo // RB
    b1 = (hi + RB - 1) // RB
    c0 = cb[pl.ds(0, 16)]
    c1 = cb[pl.ds(16, 16)]
    lane = lax.iota(jnp.int32, 16)

    zeros16 = jnp.zeros((16,), jnp.float32)
    for k in range(ACC // 16):
        acc[pl.ds(16 * k, 16)] = zeros16

    def _issue(i, par, sem):
        pltpu.async_copy(feat_hbm.at[pl.ds(i * RB, RB), :],
                         fbuf.at[pl.ds(par * RB, RB), :], sem)
        pltpu.async_copy(p_hbm.at[pl.ds(i * RB, RB)],
                         pbuf.at[pl.ds(par * RB, RB)], sem)
        pltpu.async_copy(seg_hbm.at[pl.ds(i * RB, RB)],
                         sbuf.at[pl.ds(par * RB, RB)], sem)

    def _drain(i, par, sem):
        pltpu.make_async_copy(feat_hbm.at[pl.ds(i * RB, RB), :],
                              fbuf.at[pl.ds(par * RB, RB), :], sem).wait()
        pltpu.make_async_copy(p_hbm.at[pl.ds(i * RB, RB)],
                              pbuf.at[pl.ds(par * RB, RB)], sem).wait()
        pltpu.make_async_copy(seg_hbm.at[pl.ds(i * RB, RB)],
                              sbuf.at[pl.ds(par * RB, RB)], sem).wait()

    @pl.when(b0 < b1)
    def _():
        _issue(b0, 0, sem0)

    def buf_body(i, carry):
        par = lax.rem(i - b0, 2)
        nxt = i + 1

        @pl.when(nxt < b1)
        def _():
            @pl.when(par == 0)
            def _():
                _issue(nxt, 1, sem1)

            @pl.when(par == 1)
            def _():
                _issue(nxt, 0, sem0)

        @pl.when(par == 0)
        def _():
            _drain(i, 0, sem0)

        @pl.when(par == 1)
        def _():
            _drain(i, 1, sem1)

        bi = jnp.full((16,), i // BPB)
        corr_spl = jnp.where(
            bi < 16,
            c0.at[jnp.clip(bi, 0, 15)].get(mode='promise_in_bounds'),
            c1.at[jnp.clip(bi - 16, 0, 15)].get(mode='promise_in_bounds'))

        def sb_body(sb, carry2):
            run2, sd, dvec = carry2[0], carry2[1], carry2[2]
            a = list(carry2[3:])
            rbase = par * RB + sb * 16
            svec = sbuf[pl.ds(rbase, 16)]
            pvec = pbuf[pl.ds(rbase, 16)] * corr_spl
            for r in range(16):
                s_r = svec[r]
                p_r = pvec[r]
                ok = (s_r >= lo8) & (s_r < lo8 + SEG_PER_W)
                eff = jnp.where(ok, s_r, -1)
                changed = eff != run2
                do_flush = changed & (run2 >= 0)
                abase = (run2 - lo8) * D

                @pl.when(do_flush)
                def _():
                    for j in range(NC16):
                        acc[pl.ds(abase + 16 * j, 16)] = a[j]

                dvec = jnp.where(do_flush & (lane == run2 - lo8),
                                 jnp.full((16,), sd), dvec)
                contrib = jnp.where(ok, p_r, 0.0)
                zf = changed & ok
                sd = jnp.where(zf, 0.0, sd) + contrib
                row = rbase + r
                a = [jnp.where(zf, 0.0, a[j]) +
                     contrib * fbuf[row, pl.ds(16 * j, 16)]
                     for j in range(NC16)]
                run2 = eff
            return tuple([run2, sd, dvec] + a)

        return lax.fori_loop(0, SB, sb_body, carry)

    init = tuple([jnp.int32(-1), jnp.float32(0.0), zeros16] +
                 [zeros16] * NC16)
    fin = lax.fori_loop(b0, b1, buf_body, init)
    run_f, sd_f, dvec_f = fin[0], fin[1], fin[2]
    abase_f = (run_f - lo8) * D

    @pl.when(run_f >= 0)
    def _():
        for j in range(NC16):
            acc[pl.ds(abase_f + 16 * j, 16)] = fin[3 + j]

    dvec_f = jnp.where((run_f >= 0) & (lane == run_f - lo8),
                       jnp.full((16,), sd_f), dvec_f)
    dvm[...] = dvec_f
    pltpu.sync_copy(acc, out_hbm.at[pl.ds(w * ACC, ACC)])
    pltpu.sync_copy(dvm, den_hbm.at[pl.ds(w * 16, 16)])


def _final_kernel(pooled_lo, pooled_hi, den_lo, den_hi, wn, bn, out):
    d = jnp.concatenate([den_lo[0, :], den_hi[0, :]])[:, None]
    msk = d > 0.0
    inv = jnp.where(msk, 1.0 / jnp.where(msk, d, 1.0), 0.0)
    pn = jnp.concatenate([pooled_lo[...], pooled_hi[...]], axis=0) * inv
    out[...] = jnp.dot(pn, wn[...], preferred_element_type=jnp.float32) + \
        jnp.where(msk, bn[...], 0.0)


def kernel(feat, segment_ids, Wg, bg, Wn, bn):
    seg32 = segment_ids.astype(jnp.int32)
    seg_s = seg32[::16].reshape(NB, 1, B // 16)
    seg3 = seg32.reshape(NB, 1, B)
    bg2 = bg.reshape(1, 1)
    bn2 = bn.reshape(1, D)

    p, cnt, corr, pooled_lo, den_lo = pl.pallas_call(
        _gate_kernel,
        grid=(NB,),
        in_specs=[
            pl.BlockSpec((B, D), lambda i: (i, 0)),
            pl.BlockSpec((1, 1, B // 16), lambda i: (i, 0, 0)),
            pl.BlockSpec((1, 1, B), lambda i: (i, 0, 0)),
            pl.BlockSpec((D, 1), lambda i: (0, 0)),
            pl.BlockSpec((1, 1), lambda i: (0, 0)),
        ],
        out_specs=[
            pl.BlockSpec((B, 1), lambda i: (i, 0)),
            pl.BlockSpec((1, G), lambda i: (0, 0)),
            pl.BlockSpec((1, 32), lambda i: (0, 0)),
            pl.BlockSpec((GH, D), lambda i: (0, 0)),
            pl.BlockSpec((1, GH), lambda i: (0, 0)),
        ],
        out_shape=[
            jax.ShapeDtypeStruct((N, 1), jnp.float32),
            jax.ShapeDtypeStruct((1, G), jnp.float32),
            jax.ShapeDtypeStruct((1, 32), jnp.float32),
            jax.ShapeDtypeStruct((GH, D), jnp.float32),
            jax.ShapeDtypeStruct((1, GH), jnp.float32),
        ],
        scratch_shapes=[pltpu.VMEM((1, 32), jnp.float32),
                        pltpu.VMEM((1, GH), jnp.float32)],
    )(feat, seg_s, seg3, Wg, bg2)

    si = cnt[0].astype(jnp.int32)
    ks = GH + SEG_PER_W * jnp.arange(NW)
    s_lo = jnp.take(si, ks)
    s_hi = jnp.take(jnp.concatenate([si, jnp.array([N // 16], jnp.int32)]),
                    ks + SEG_PER_W)
    lo_w = jnp.maximum(16 * s_lo - 16, 0)
    hi_w = jnp.minimum(16 * s_hi, N)
    zc = jnp.zeros((NW,), jnp.int32)
    bounds = jnp.stack([lo_w, hi_w, zc, zc, zc, zc, zc, zc],
                       axis=1).reshape(NW * 8)
    bounds = jnp.pad(bounds, (0, 8))

    sc_pool = pl.kernel(
        _sc_pool_kernel,
        out_type=[
            jax.ShapeDtypeStruct((GH * D,), jnp.float32),
            jax.ShapeDtypeStruct((NW * 16,), jnp.float32),
        ],
        mesh=plsc.VectorSubcoreMesh(core_axis_name="c", subcore_axis_name="s"),
        scratch_types=[
            pltpu.VMEM((16,), jnp.int32),
            pltpu.VMEM((2 * RB, D), jnp.float32),
            pltpu.VMEM((2 * RB,), jnp.float32),
            pltpu.VMEM((2 * RB,), jnp.int32),
            pltpu.VMEM((ACC,), jnp.float32),
            pltpu.VMEM((32,), jnp.float32),
            pltpu.VMEM((16,), jnp.float32),
            pltpu.SemaphoreType.DMA,
            pltpu.SemaphoreType.DMA,
        ],
        compiler_params=pltpu.CompilerParams(needs_layout_passes=False),
    )
    pooled_hi_flat, den_raw = sc_pool(feat, p.reshape(N), seg32, bounds,
                                      corr.reshape(32))
    pooled_hi = pooled_hi_flat.reshape(GH, D)
    den_hi = den_raw.reshape(NW, 16)[:, :SEG_PER_W].reshape(1, GH)

    out = pl.pallas_call(
        _final_kernel,
        in_specs=[
            pl.BlockSpec((GH, D), lambda: (0, 0)),
            pl.BlockSpec((GH, D), lambda: (0, 0)),
            pl.BlockSpec((1, GH), lambda: (0, 0)),
            pl.BlockSpec((1, GH), lambda: (0, 0)),
            pl.BlockSpec((D, D), lambda: (0, 0)),
            pl.BlockSpec((1, D), lambda: (0, 0)),
        ],
        out_specs=pl.BlockSpec((G, D), lambda: (0, 0)),
        out_shape=jax.ShapeDtypeStruct((G, D), jnp.float32),
    )(pooled_lo, pooled_hi, den_lo, den_hi, Wn, bn2)

    return out


# speedup vs baseline: 2.3070x; 1.0938x over previous
import functools
import jax
import jax.numpy as jnp
from jax import lax
from jax.experimental import pallas as pl
from jax.experimental.pallas import tpu as pltpu
from jax.experimental.pallas import tpu_sc as plsc

N = 50000
D = 512
G = 256
B = 2000
NB = N // B

NW = 32
GH = 192
GS = G - GH
SEG_PER_W = GS // NW
RB = 80
NBUF = N // RB
SB = RB // 16
FB = RB * D
ACC = SEG_PER_W * D
NC16 = D // 16
BPB = B // RB

_NEG = -1e30


def _gate_kernel(feat, seg_s, seg, wg, bg, p_out, cnt_out, corr_out,
                 pooled_lo, den_lo, m_scr, mrun):
    i = pl.program_id(0)

    @pl.when(i == 0)
    def _():
        cnt_out[...] = jnp.zeros_like(cnt_out)
        m_scr[...] = jnp.full_like(m_scr, _NEG)
        mrun[...] = jnp.full_like(mrun, _NEG)
        pooled_lo[...] = jnp.zeros_like(pooled_lo)
        den_lo[...] = jnp.zeros_like(den_lo)

    x = feat[...]
    g = jnp.dot(x, wg[...], preferred_element_type=jnp.float32) + bg[0, 0]
    lm = jnp.max(g)
    pv = jnp.exp(g - lm)
    p_out[...] = pv
    ids32 = lax.broadcasted_iota(jnp.int32, (1, 32), 1)
    m_scr[0, :] = jnp.where(ids32[0, :] == i, lm, m_scr[0, :])
    ss = seg_s[0, 0, :]
    ids = lax.broadcasted_iota(jnp.int32, (B // 16, G), 1)
    below = (ss[:, None] < ids).astype(jnp.float32)
    cnt_out[0, :] = cnt_out[0, :] + jnp.sum(below, axis=0)

    mo = mrun[0, :]
    mn = jnp.maximum(mo, lm)
    f_s = jnp.exp(mo - mn)[0]
    e_b = jnp.exp(lm - mn[0])
    s_full = seg[0, 0, :]
    ids_lo = lax.broadcasted_iota(jnp.int32, (B, GH), 1)
    oh_lo = (s_full[:, None] == ids_lo).astype(jnp.float32)
    pw = pv * e_b
    pooled_lo[...] = pooled_lo[...] * f_s + lax.dot_general(
        oh_lo, x * pw, dimension_numbers=(((0,), (0,)), ((), ())),
        preferred_element_type=jnp.float32)
    den_lo[0, :] = den_lo[0, :] * f_s + jnp.sum(oh_lo * pw, axis=0)
    mrun[0, :] = mn

    @pl.when(i == NB - 1)
    def _():
        mv = m_scr[0, :]
        mx = jnp.max(mv)
        corr_out[0, :] = jnp.exp(jnp.minimum(mv - mx, 0.0))


def _sc_pool_kernel(feat_hbm, p_hbm, seg_hbm, bounds_hbm, corr_hbm,
                    out_hbm, den_hbm,
                    sv, fbuf, pbuf, sbuf, acc, cb, dvm, sem0, sem1):
    c = lax.axis_index("c")
    s = lax.axis_index("s")
    w = s * 2 + c
    lo8 = GH + w * SEG_PER_W
    pltpu.sync_copy(bounds_hbm.at[pl.ds(w * 8, 16)], sv)
    pltpu.sync_copy(corr_hbm, cb)
    svv = sv[...]
    lo = svv[0]
    hi = svv[1]
    b0 = lo // RB
    b1 = (hi + RB - 1) // RB
    c0 = cb[pl.ds(0, 16)]
    c1 = cb[pl.ds(16, 16)]
    lane = lax.iota(jnp.int32, 16)

    zeros16 = jnp.zeros((16,), jnp.float32)
    for k in range(ACC // 16):
        acc[pl.ds(16 * k, 16)] = zeros16

    def _issue(i, par, sem):
        pltpu.async_copy(feat_hbm.at[pl.ds(i * RB, RB), :],
                         fbuf.at[pl.ds(par * RB, RB), :], sem)
        pltpu.async_copy(p_hbm.at[pl.ds(i * RB, RB)],
                         pbuf.at[pl.ds(par * RB, RB)], sem)
        pltpu.async_copy(seg_hbm.at[pl.ds(i * RB, RB)],
                         sbuf.at[pl.ds(par * RB, RB)], sem)

    def _drain(i, par, sem):
        pltpu.make_async_copy(feat_hbm.at[pl.ds(i * RB, RB), :],
                              fbuf.at[pl.ds(par * RB, RB), :], sem).wait()
        pltpu.make_async_copy(p_hbm.at[pl.ds(i * RB, RB)],
                              pbuf.at[pl.ds(par * RB, RB)], sem).wait()
        pltpu.make_async_copy(seg_hbm.at[pl.ds(i * RB, RB)],
                              sbuf.at[pl.ds(par * RB, RB)], sem).wait()

    @pl.when(b0 < b1)
    def _():
        _issue(b0, 0, sem0)

    def buf_body(i, carry):
        par = lax.rem(i - b0, 2)
        nxt = i + 1

        @pl.when(nxt < b1)
        def _():
            @pl.when(par == 0)
            def _():
                _issue(nxt, 1, sem1)

            @pl.when(par == 1)
            def _():
                _issue(nxt, 0, sem0)

        @pl.when(par == 0)
        def _():
            _drain(i, 0, sem0)

        @pl.when(par == 1)
        def _():
            _drain(i, 1, sem1)

        bi = jnp.full((16,), i // BPB)
        corr_spl = jnp.where(
            bi < 16,
            c0.at[jnp.clip(bi, 0, 15)].get(mode='promise_in_bounds'),
            c1.at[jnp.clip(bi - 16, 0, 15)].get(mode='promise_in_bounds'))

        def sb_body(sb, carry2):
            run2, sd, dvec = carry2[0], carry2[1], carry2[2]
            a = list(carry2[3:])
            rbase = par * RB + sb * 16
            svec = sbuf[pl.ds(rbase, 16)]
            pvec = pbuf[pl.ds(rbase, 16)] * corr_spl
            for r in range(16):
                s_r = svec[r]
                p_r = pvec[r]
                ok = (s_r >= lo8) & (s_r < lo8 + SEG_PER_W)
                eff = jnp.where(ok, s_r, -1)
                changed = eff != run2
                do_flush = changed & (run2 >= 0)
                abase = (run2 - lo8) * D

                @pl.when(do_flush)
                def _():
                    for j in range(NC16):
                        acc[pl.ds(abase + 16 * j, 16)] = a[j]

                dvec = jnp.where(do_flush & (lane == run2 - lo8),
                                 jnp.full((16,), sd), dvec)
                contrib = jnp.where(ok, p_r, 0.0)
                zf = changed & ok
                sd = jnp.where(zf, 0.0, sd) + contrib
                row = rbase + r
                a = [jnp.where(zf, 0.0, a[j]) +
                     contrib * fbuf[row, pl.ds(16 * j, 16)]
                     for j in range(NC16)]
                run2 = eff
            return tuple([run2, sd, dvec] + a)

        return lax.fori_loop(0, SB, sb_body, carry)

    init = tuple([jnp.int32(-1), jnp.float32(0.0), zeros16] +
                 [zeros16] * NC16)
    fin = lax.fori_loop(b0, b1, buf_body, init)
    run_f, sd_f, dvec_f = fin[0], fin[1], fin[2]
    abase_f = (run_f - lo8) * D

    @pl.when(run_f >= 0)
    def _():
        for j in range(NC16):
            acc[pl.ds(abase_f + 16 * j, 16)] = fin[3 + j]

    dvec_f = jnp.where((run_f >= 0) & (lane == run_f - lo8),
                       jnp.full((16,), sd_f), dvec_f)
    dvm[...] = dvec_f
    pltpu.sync_copy(acc, out_hbm.at[pl.ds(w * ACC, ACC)])
    pltpu.sync_copy(dvm, den_hbm.at[pl.ds(w * 16, 16)])


def _final_kernel(pooled_lo, pooled_hi, den_lo, den_hi, wn, bn, out):
    d = jnp.concatenate([den_lo[0, :], den_hi[0, :]])[:, None]
    msk = d > 0.0
    inv = jnp.where(msk, 1.0 / jnp.where(msk, d, 1.0), 0.0)
    pn = jnp.concatenate([pooled_lo[...], pooled_hi[...]], axis=0) * inv
    out[...] = jnp.dot(pn, wn[...], preferred_element_type=jnp.float32) + \
        jnp.where(msk, bn[...], 0.0)


def kernel(feat, segment_ids, Wg, bg, Wn, bn):
    seg32 = segment_ids.astype(jnp.int32)
    seg_s = seg32[::16].reshape(NB, 1, B // 16)
    seg3 = seg32.reshape(NB, 1, B)
    bg2 = bg.reshape(1, 1)
    bn2 = bn.reshape(1, D)

    p, cnt, corr, pooled_lo, den_lo = pl.pallas_call(
        _gate_kernel,
        grid=(NB,),
        in_specs=[
            pl.BlockSpec((B, D), lambda i: (i, 0)),
            pl.BlockSpec((1, 1, B // 16), lambda i: (i, 0, 0)),
            pl.BlockSpec((1, 1, B), lambda i: (i, 0, 0)),
            pl.BlockSpec((D, 1), lambda i: (0, 0)),
            pl.BlockSpec((1, 1), lambda i: (0, 0)),
        ],
        out_specs=[
            pl.BlockSpec((B, 1), lambda i: (i, 0)),
            pl.BlockSpec((1, G), lambda i: (0, 0)),
            pl.BlockSpec((1, 32), lambda i: (0, 0)),
            pl.BlockSpec((GH, D), lambda i: (0, 0)),
            pl.BlockSpec((1, GH), lambda i: (0, 0)),
        ],
        out_shape=[
            jax.ShapeDtypeStruct((N, 1), jnp.float32),
            jax.ShapeDtypeStruct((1, G), jnp.float32),
            jax.ShapeDtypeStruct((1, 32), jnp.float32),
            jax.ShapeDtypeStruct((GH, D), jnp.float32),
            jax.ShapeDtypeStruct((1, GH), jnp.float32),
        ],
        scratch_shapes=[pltpu.VMEM((1, 32), jnp.float32),
                        pltpu.VMEM((1, GH), jnp.float32)],
    )(feat, seg_s, seg3, Wg, bg2)

    si = cnt[0].astype(jnp.int32)
    ks = GH + SEG_PER_W * jnp.arange(NW)
    s_lo = jnp.take(si, ks)
    s_hi = jnp.take(jnp.concatenate([si, jnp.array([N // 16], jnp.int32)]),
                    ks + SEG_PER_W)
    lo_w = jnp.maximum(16 * s_lo - 16, 0)
    hi_w = jnp.minimum(16 * s_hi, N)
    zc = jnp.zeros((NW,), jnp.int32)
    bounds = jnp.stack([lo_w, hi_w, zc, zc, zc, zc, zc, zc],
                       axis=1).reshape(NW * 8)
    bounds = jnp.pad(bounds, (0, 8))

    sc_pool = pl.kernel(
        _sc_pool_kernel,
        out_type=[
            jax.ShapeDtypeStruct((GS * D,), jnp.float32),
            jax.ShapeDtypeStruct((NW * 16,), jnp.float32),
        ],
        mesh=plsc.VectorSubcoreMesh(core_axis_name="c", subcore_axis_name="s"),
        scratch_types=[
            pltpu.VMEM((16,), jnp.int32),
            pltpu.VMEM((2 * RB, D), jnp.float32),
            pltpu.VMEM((2 * RB,), jnp.float32),
            pltpu.VMEM((2 * RB,), jnp.int32),
            pltpu.VMEM((ACC,), jnp.float32),
            pltpu.VMEM((32,), jnp.float32),
            pltpu.VMEM((16,), jnp.float32),
            pltpu.SemaphoreType.DMA,
            pltpu.SemaphoreType.DMA,
        ],
        compiler_params=pltpu.CompilerParams(needs_layout_passes=False),
    )
    pooled_hi_flat, den_raw = sc_pool(feat, p.reshape(N), seg32, bounds,
                                      corr.reshape(32))
    pooled_hi = pooled_hi_flat.reshape(GS, D)
    den_hi = den_raw.reshape(NW, 16)[:, :SEG_PER_W].reshape(1, GS)

    out = pl.pallas_call(
        _final_kernel,
        in_specs=[
            pl.BlockSpec((GH, D), lambda: (0, 0)),
            pl.BlockSpec((GS, D), lambda: (0, 0)),
            pl.BlockSpec((1, GH), lambda: (0, 0)),
            pl.BlockSpec((1, GS), lambda: (0, 0)),
            pl.BlockSpec((D, D), lambda: (0, 0)),
            pl.BlockSpec((1, D), lambda: (0, 0)),
        ],
        out_specs=pl.BlockSpec((G, D), lambda: (0, 0)),
        out_shape=jax.ShapeDtypeStruct((G, D), jnp.float32),
    )(pooled_lo, pooled_hi, den_lo, den_hi, Wn, bn2)

    return out
